# baseline (speedup 1.0000x reference)
import functools
import numpy as np
import jax
import jax.numpy as jnp
from jax import lax
from jax.experimental import pallas as pl
from jax.experimental.pallas import tpu as pltpu
from jax.experimental.pallas import tpu_sc as plsc

D_MODEL = 256
N_HEADS = 8
D_H = D_MODEL // N_HEADS
QUARTER = D_H // 4
D_FF = 1024
N_LAYERS = 2
DIAMETER = 7
KNBR = DIAMETER * DIAMETER
THETA = 10.0
R = DIAMETER // 2


def _np_constants():
    f = THETA ** (-np.arange(QUARTER, dtype=np.float32) / QUARTER)
    F2 = np.zeros((2, 2 * QUARTER), np.float32)
    F2[0, :QUARTER] = f
    F2[1, QUARTER:] = f
    d = np.arange(D_MODEL)
    jmod = d % (2 * QUARTER)
    EC = (jmod[None, :] == np.arange(2 * QUARTER)[:, None]).astype(np.float32)
    sgn = np.where((d % D_H) < (2 * QUARTER), -1.0, 1.0).astype(np.float32)
    ES = EC * sgn[None, :]
    G = (d[:, None] // D_H == np.arange(N_HEADS)[None, :]).astype(np.float32)
    s = np.arange(KNBR)
    Doff = np.stack([s // DIAMETER - R, s % DIAMETER - R], axis=-1).astype(np.float32)
    return F2, EC, ES, G, Doff


_F2, _EC, _ES, _G, _DOFF = _np_constants()


def _swap_halves(x):
    parts = []
    for h in range(N_HEADS):
        parts.append(x[:, h * D_H + 2 * QUARTER:(h + 1) * D_H])
        parts.append(x[:, h * D_H:h * D_H + 2 * QUARTER])
    return jnp.concatenate(parts, axis=-1)


def _rope(x, pos2, F2, EC, ES):
    ang16 = jnp.dot(pos2, F2, preferred_element_type=jnp.float32)
    c16 = jnp.cos(ang16)
    s16 = jnp.sin(ang16)
    cf = jnp.dot(c16, EC, preferred_element_type=jnp.float32)
    sf = jnp.dot(s16, ES, preferred_element_type=jnp.float32)
    return x * cf + _swap_halves(x) * sf


def _ln(x, g, b):
    m = jnp.mean(x, axis=-1, keepdims=True)
    d = x - m
    v = jnp.mean(d * d, axis=-1, keepdims=True)
    return d * lax.rsqrt(v + 1e-5) * g + b


def _prep_body(pos_ref, offs_ref, lss_ref, idx_ref):
    Q = pos_ref.shape[0]
    li = jnp.int32(0)
    best = lss_ref[0, 0] * lss_ref[0, 1]
    n_lvl = lss_ref.shape[0]
    for j in range(1, n_lvl):
        p = lss_ref[j, 0] * lss_ref[j, 1]
        li = jnp.where(p > best, jnp.int32(j), li)
        best = jnp.maximum(p, best)
    q_ids = lax.broadcasted_iota(jnp.int32, (Q, KNBR), 0)
    s_ids = lax.broadcasted_iota(jnp.int32, (Q, KNBR), 1)
    bid = jnp.zeros((Q, KNBR), jnp.int32)
    for j in range(offs_ref.shape[0]):
        bid = bid + (q_ids >= offs_ref[j]).astype(jnp.int32)
    bid = bid - 1
    cy = jnp.floor(pos_ref[:, 0:1]).astype(jnp.int32)
    cx = jnp.floor(pos_ref[:, 1:2]).astype(jnp.int32)
    ky = cy + (s_ids // DIAMETER - R)
    kx = cx + (s_ids % DIAMETER - R)
    H = 64
    W = 64
    NLVL = 4
    kyc = jnp.clip(ky, 0, H - 1)
    kxc = jnp.clip(kx, 0, W - 1)
    idx_ref[...] = ((bid * H + kyc) * W + kxc) * NLVL + li


def _prep_indices(query_positions, query_batch_offsets, level_spatial_shapes):
    Q = query_positions.shape[0]
    return pl.pallas_call(
        _prep_body,
        out_shape=jax.ShapeDtypeStruct((Q, KNBR), jnp.int32),
        in_specs=[
            pl.BlockSpec(memory_space=pltpu.VMEM),
            pl.BlockSpec(memory_space=pltpu.SMEM),
            pl.BlockSpec(memory_space=pltpu.SMEM),
        ],
        out_specs=pl.BlockSpec(memory_space=pltpu.VMEM),
    )(query_positions, query_batch_offsets, level_spatial_shapes)


def _sc_gather(table, idx_flat):
    rows = idx_flat.shape[0]
    info = plsc.get_sparse_core_info()
    nw = info.num_cores * info.num_subcores
    per_w = rows // nw
    CH = 224
    nchunks = per_w // CH
    mesh = plsc.VectorSubcoreMesh(core_axis_name="c", subcore_axis_name="s")

    @functools.partial(
        pl.kernel,
        mesh=mesh,
        out_type=jax.ShapeDtypeStruct((rows, D_MODEL), jnp.float32),
        scratch_types=[
            pltpu.VMEM((per_w,), jnp.int32),
            pltpu.VMEM((CH, D_MODEL), jnp.float32),
            pltpu.VMEM((CH, D_MODEL), jnp.float32),
            pltpu.SemaphoreType.DMA,
            pltpu.SemaphoreType.DMA,
        ],
    )
    def gather_kernel(table_hbm, idx_hbm, out_hbm, idx_v, rows0, rows1, sem0, sem1):
        wid = lax.axis_index("s") * info.num_cores + lax.axis_index("c")
        base = wid * per_w
        pltpu.sync_copy(idx_hbm.at[pl.ds(base, per_w)], idx_v)
        bufs = [rows0, rows1]
        sems = [sem0, sem1]
        copies = [None, None]
        copies[0] = pltpu.async_copy(
            table_hbm.at[idx_v.at[pl.ds(0, CH)]], bufs[0], sems[0])
        for c in range(nchunks):
            nxt = c + 1
            if nxt < nchunks:
                copies[nxt % 2] = pltpu.async_copy(
                    table_hbm.at[idx_v.at[pl.ds(nxt * CH, CH)]],
                    bufs[nxt % 2], sems[nxt % 2])
            copies[c % 2].wait()
            pltpu.sync_copy(bufs[c % 2], out_hbm.at[pl.ds(base + c * CH, CH)])

    return gather_kernel(table, idx_flat)


CQ = 8


def _main_body(x_ref, pos_ref, feats_ref, F2_ref, EC_ref, ES_ref, G_ref,
               Doff_ref, *rest):
    out_ref, q_s, att_s = rest[-3], rest[-2], rest[-1]
    wrefs = rest[:-3]
    BQ = x_ref.shape[0]
    NC = BQ // CQ
    CR = CQ * KNBR

    F2 = F2_ref[...]
    EC = EC_ref[...]
    ES = ES_ref[...]
    G = G_ref[...]
    Doff = Doff_ref[...]

    pos = pos_ref[...]
    x = x_ref[...]

    def chunk_geom(c):
        pos_c = pos_ref[pl.ds(c * CQ, CQ), :]
        centers = jnp.floor(pos_c)
        crep = jnp.broadcast_to(centers[:, None, :], (CQ, KNBR, 2)).reshape(CR, 2)
        KY = crep + jnp.broadcast_to(Doff[None], (CQ, KNBR, 2)).reshape(CR, 2)
        ky = KY[:, 0:1]
        kx = KY[:, 1:2]
        oob = (ky < 0.0) | (ky > 63.0) | (kx < 0.0) | (kx > 63.0)
        return KY, oob

    wi = 0
    for _ in range(N_LAYERS):
        ln1_g, ln1_b, Wq, Wk, Wv, Wo, ln2_g, ln2_b, W1, b1, W2, b2 = (
            wrefs[wi + j][...] for j in range(12))
        wi += 12

        h = _ln(x, ln1_g, ln1_b)
        q = jnp.dot(h, Wq, preferred_element_type=jnp.float32)
        q_s[...] = _rope(q, pos, F2, EC, ES)

        def attn_chunk(c, carry):
            KY, oob = chunk_geom(c)
            fs = feats_ref[pl.ds(c * CR, CR), :]
            k = jnp.dot(fs, Wk, preferred_element_type=jnp.float32)
            k = _rope(k, KY, F2, EC, ES)
            v = jnp.dot(fs, Wv, preferred_element_type=jnp.float32)
            qc = q_s[pl.ds(c * CQ, CQ), :]
            qrep = jnp.broadcast_to(qc[:, None, :], (CQ, KNBR, D_MODEL)).reshape(CR, D_MODEL)
            t = qrep * k
            scores = jnp.dot(t, G, preferred_element_type=jnp.float32) * (1.0 / np.sqrt(D_H))
            scores = jnp.where(oob, -1e9, scores)
            s3 = scores.reshape(CQ, KNBR, N_HEADS)
            smax = jnp.max(s3, axis=1, keepdims=True)
            e = jnp.exp(s3 - jnp.broadcast_to(smax, (CQ, KNBR, N_HEADS)))
            denom = jnp.sum(e, axis=1, keepdims=True)
            attn = (e / jnp.broadcast_to(denom, (CQ, KNBR, N_HEADS))).reshape(CR, N_HEADS)
            attn_full = jnp.dot(attn, G.T, preferred_element_type=jnp.float32)
            weighted = (attn_full * v).reshape(CQ, KNBR, D_MODEL)
            att_s[pl.ds(c * CQ, CQ), :] = jnp.sum(weighted, axis=1)
            return carry

        lax.fori_loop(0, NC, attn_chunk, 0, unroll=False)

        x = x + jnp.dot(att_s[...], Wo, preferred_element_type=jnp.float32)
        h2 = _ln(x, ln2_g, ln2_b)
        u = jax.nn.gelu(jnp.dot(h2, W1, preferred_element_type=jnp.float32) + b1)
        x = x + jnp.dot(u, W2, preferred_element_type=jnp.float32) + b2

    q_s[...] = x

    def logit_chunk(c, carry):
        _, oob = chunk_geom(c)
        fs = feats_ref[pl.ds(c * CR, CR), :]
        xc = q_s[pl.ds(c * CQ, CQ), :]
        xrep = jnp.broadcast_to(xc[:, None, :], (CQ, KNBR, D_MODEL)).reshape(CR, D_MODEL)
        lg = jnp.sum(xrep * fs, axis=-1, keepdims=True)
        out_ref[pl.ds(c * CR, CR), :] = jnp.where(oob, 0.0, lg)
        return carry

    lax.fori_loop(0, NC, logit_chunk, 0, unroll=False)


def _main_call(queries, query_positions, feats, wlist, BQ=128):
    Q = queries.shape[0]
    grid = Q // BQ
    consts = [jnp.asarray(_F2), jnp.asarray(_EC), jnp.asarray(_ES),
              jnp.asarray(_G), jnp.asarray(_DOFF)]
    in_specs = [
        pl.BlockSpec((BQ, D_MODEL), lambda i: (i, 0)),
        pl.BlockSpec((BQ, 2), lambda i: (i, 0)),
        pl.BlockSpec((BQ * KNBR, D_MODEL), lambda i: (i, 0)),
    ]
    for w in consts + wlist:
        nd = w.ndim
        in_specs.append(pl.BlockSpec(w.shape, lambda i, _nd=nd: (0,) * _nd))
    return pl.pallas_call(
        _main_body,
        grid=(grid,),
        out_shape=jax.ShapeDtypeStruct((Q * KNBR, 1), jnp.float32),
        in_specs=in_specs,
        out_specs=pl.BlockSpec((BQ * KNBR, 1), lambda i: (i, 0)),
        scratch_shapes=[
            pltpu.VMEM((BQ, D_MODEL), jnp.float32),
            pltpu.VMEM((BQ, D_MODEL), jnp.float32),
        ],
        compiler_params=pltpu.CompilerParams(
            dimension_semantics=("arbitrary",),
        ),
    )(queries, query_positions, feats, *consts, *wlist)


def kernel(queries, query_batch_offsets, query_positions, stacked_feature_map,
           level_spatial_shapes, params):
    Q = queries.shape[0]
    table = stacked_feature_map.reshape(-1, D_MODEL)
    idx2d = _prep_indices(query_positions, query_batch_offsets, level_spatial_shapes)
    feats = _sc_gather(table, idx2d.reshape(-1))
    wlist = []
    for i in range(N_LAYERS):
        wlist += [
            params[f"l{i}_ln1_g"].reshape(1, D_MODEL),
            params[f"l{i}_ln1_b"].reshape(1, D_MODEL),
            params[f"l{i}_Wq"], params[f"l{i}_Wk"],
            params[f"l{i}_Wv"], params[f"l{i}_Wo"],
            params[f"l{i}_ln2_g"].reshape(1, D_MODEL),
            params[f"l{i}_ln2_b"].reshape(1, D_MODEL),
            params[f"l{i}_W1"], params[f"l{i}_b1"].reshape(1, D_FF),
            params[f"l{i}_W2"], params[f"l{i}_b2"].reshape(1, D_MODEL),
        ]
    logits_col = _main_call(queries, query_positions, feats, wlist)
    return logits_col.reshape(Q, KNBR)


# speedup vs baseline: 2.3502x; 2.3502x over previous
import functools
import numpy as np
import jax
import jax.numpy as jnp
from jax import lax
from jax.experimental import pallas as pl
from jax.experimental.pallas import tpu as pltpu
from jax.experimental.pallas import tpu_sc as plsc

D_MODEL = 256
N_HEADS = 8
D_H = D_MODEL // N_HEADS
QUARTER = D_H // 4
D_FF = 1024
N_LAYERS = 2
DIAMETER = 7
KNBR = DIAMETER * DIAMETER
THETA = 10.0
R = DIAMETER // 2


CQ = 16
CR = CQ * KNBR
HALF = D_MODEL // 2


def _np_constants():
    perm = np.zeros((D_MODEL,), np.int64)
    for h in range(N_HEADS):
        for j in range(2 * QUARTER):
            perm[h * 16 + j] = h * D_H + j
            perm[HALF + h * 16 + j] = h * D_H + 2 * QUARTER + j
    f = THETA ** (-np.arange(QUARTER, dtype=np.float32) / QUARTER)
    F2 = np.zeros((2, 2 * QUARTER), np.float32)
    F2[0, :QUARTER] = f
    F2[1, QUARTER:] = f
    l = np.arange(HALF)
    E = (l[None, :] % 16 == np.arange(16)[:, None]).astype(np.float32)
    d = np.arange(D_MODEL)
    Gs = ((d[:, None] % HALF) // 16 == np.arange(N_HEADS)[None, :]).astype(np.float32)
    Gs = Gs / np.sqrt(np.float32(D_H))
    Gt = (d[None, :] // D_H == np.arange(N_HEADS)[:, None]).astype(np.float32)
    r = np.arange(CR)
    S = (r[:, None] // KNBR == np.arange(CQ)[None, :]).astype(np.float32)
    s = np.arange(KNBR)
    Doff = np.stack([s // DIAMETER - R, s % DIAMETER - R], axis=-1).astype(np.float32)
    Dtile = np.tile(Doff, (CQ, 1))
    ones = np.ones((D_MODEL, 1), np.float32)
    return perm, F2, E, Gs, Gt, S, S.T.copy(), Dtile, ones


_PERM, _F2, _E, _GS, _GT, _S, _ST, _DTILE, _ONES = _np_constants()


def _rope_planar(x, ce, se):
    xr = x[:, :HALF]
    xi = x[:, HALF:]
    return jnp.concatenate([xr * ce - xi * se, xr * se + xi * ce], axis=-1)


def _ln(x, g, b):
    m = jnp.mean(x, axis=-1, keepdims=True)
    d = x - m
    v = jnp.mean(d * d, axis=-1, keepdims=True)
    return d * lax.rsqrt(v + 1e-5) * g + b


def _prep_body(pos_ref, offs_ref, lss_ref, idx_ref):
    Q = pos_ref.shape[0]
    li = jnp.int32(0)
    best = lss_ref[0, 0] * lss_ref[0, 1]
    n_lvl = lss_ref.shape[0]
    for j in range(1, n_lvl):
        p = lss_ref[j, 0] * lss_ref[j, 1]
        li = jnp.where(p > best, jnp.int32(j), li)
        best = jnp.maximum(p, best)
    q_ids = lax.broadcasted_iota(jnp.int32, (Q, KNBR), 0)
    s_ids = lax.broadcasted_iota(jnp.int32, (Q, KNBR), 1)
    bid = jnp.zeros((Q, KNBR), jnp.int32)
    for j in range(offs_ref.shape[0]):
        bid = bid + (q_ids >= offs_ref[j]).astype(jnp.int32)
    bid = bid - 1
    cy = jnp.floor(pos_ref[:, 0:1]).astype(jnp.int32)
    cx = jnp.floor(pos_ref[:, 1:2]).astype(jnp.int32)
    ky = cy + (s_ids // DIAMETER - R)
    kx = cx + (s_ids % DIAMETER - R)
    H = 64
    W = 64
    NLVL = 4
    kyc = jnp.clip(ky, 0, H - 1)
    kxc = jnp.clip(kx, 0, W - 1)
    idx_ref[...] = ((bid * H + kyc) * W + kxc) * NLVL + li


def _prep_indices(query_positions, query_batch_offsets, level_spatial_shapes):
    Q = query_positions.shape[0]
    return pl.pallas_call(
        _prep_body,
        out_shape=jax.ShapeDtypeStruct((Q, KNBR), jnp.int32),
        in_specs=[
            pl.BlockSpec(memory_space=pltpu.VMEM),
            pl.BlockSpec(memory_space=pltpu.SMEM),
            pl.BlockSpec(memory_space=pltpu.SMEM),
        ],
        out_specs=pl.BlockSpec(memory_space=pltpu.VMEM),
    )(query_positions, query_batch_offsets, level_spatial_shapes)


def _sc_gather(table, idx_flat):
    rows = idx_flat.shape[0]
    info = plsc.get_sparse_core_info()
    nw = info.num_cores * info.num_subcores
    per_w = rows // nw
    CH = 224
    nchunks = per_w // CH
    mesh = plsc.VectorSubcoreMesh(core_axis_name="c", subcore_axis_name="s")

    @functools.partial(
        pl.kernel,
        mesh=mesh,
        out_type=jax.ShapeDtypeStruct((rows, D_MODEL), jnp.float32),
        scratch_types=[
            pltpu.VMEM((per_w,), jnp.int32),
            pltpu.VMEM((CH, D_MODEL), jnp.float32),
            pltpu.VMEM((CH, D_MODEL), jnp.float32),
            pltpu.SemaphoreType.DMA,
            pltpu.SemaphoreType.DMA,
        ],
    )
    def gather_kernel(table_hbm, idx_hbm, out_hbm, idx_v, rows0, rows1, sem0, sem1):
        wid = lax.axis_index("s") * info.num_cores + lax.axis_index("c")
        base = wid * per_w
        pltpu.sync_copy(idx_hbm.at[pl.ds(base, per_w)], idx_v)
        bufs = [rows0, rows1]
        sems = [sem0, sem1]
        copies = [None, None]
        copies[0] = pltpu.async_copy(
            table_hbm.at[idx_v.at[pl.ds(0, CH)]], bufs[0], sems[0])
        for c in range(nchunks):
            nxt = c + 1
            if nxt < nchunks:
                copies[nxt % 2] = pltpu.async_copy(
                    table_hbm.at[idx_v.at[pl.ds(nxt * CH, CH)]],
                    bufs[nxt % 2], sems[nxt % 2])
            copies[c % 2].wait()
            pltpu.sync_copy(bufs[c % 2], out_hbm.at[pl.ds(base + c * CH, CH)])

    return gather_kernel(table, idx_flat)


def _main_body(x_ref, pos_ref, feats_ref, F2_ref, E_ref, Gs_ref, Gt_ref,
               S_ref, St_ref, Dt_ref, ones_ref, *rest):
    out_ref, q_s, att_s = rest[-3], rest[-2], rest[-1]
    wrefs = rest[:-3]
    BQ = x_ref.shape[0]
    NC = BQ // CQ

    F2 = F2_ref[...]
    E = E_ref[...]
    Gs = Gs_ref[...]
    Gt = Gt_ref[...]
    S = S_ref[...]
    St = St_ref[...]
    Dtile = Dt_ref[...]
    ones = ones_ref[...]

    pos = pos_ref[...]
    x = x_ref[...]

    def mm(a, b):
        return jnp.dot(a, b, preferred_element_type=jnp.float32)

    def chunk_geom(c):
        pos_c = pos_ref[pl.ds(c * CQ, CQ), :]
        KY = mm(S, jnp.floor(pos_c)) + Dtile
        ky = KY[:, 0:1]
        kx = KY[:, 1:2]
        oob = (ky < 0.0) | (ky > 63.0) | (kx < 0.0) | (kx > 63.0)
        return KY, oob

    angq = mm(pos, F2)
    ceq = mm(jnp.cos(angq), E)
    seq = mm(jnp.sin(angq), E)

    wi = 0
    for _ in range(N_LAYERS):
        ln1_g, ln1_b, Wq, Wk, Wv, Wo, ln2_g, ln2_b, W1, b1, W2, b2 = (
            wrefs[wi + j][...] for j in range(12))
        wi += 12

        h = _ln(x, ln1_g, ln1_b)
        q_s[...] = _rope_planar(mm(h, Wq), ceq, seq)

        def attn_chunk(c, carry):
            KY, oob = chunk_geom(c)
            fs = feats_ref[pl.ds(c * CR, CR), :]
            angk = mm(KY, F2)
            cek = mm(jnp.cos(angk), E)
            sek = mm(jnp.sin(angk), E)
            k = _rope_planar(mm(fs, Wk), cek, sek)
            v = mm(fs, Wv)
            qrep = mm(S, q_s[pl.ds(c * CQ, CQ), :])
            scores = mm(qrep * k, Gs)
            e = jnp.exp(jnp.where(oob, -1e9, scores))
            dinv = 1.0 / mm(St, e)
            attn = e * mm(S, dinv)
            att_s[pl.ds(c * CQ, CQ), :] = mm(St, mm(attn, Gt) * v)
            return carry

        lax.fori_loop(0, NC, attn_chunk, 0, unroll=False)

        x = x + mm(att_s[...], Wo)
        h2 = _ln(x, ln2_g, ln2_b)
        u = jax.nn.gelu(mm(h2, W1) + b1)
        x = x + mm(u, W2) + b2

    q_s[...] = x

    def logit_chunk(c, carry):
        _, oob = chunk_geom(c)
        fs = feats_ref[pl.ds(c * CR, CR), :]
        xrep = mm(S, q_s[pl.ds(c * CQ, CQ), :])
        lg = mm(xrep * fs, ones)
        out_ref[pl.ds(c * CR, CR), :] = jnp.where(oob, 0.0, lg)
        return carry

    lax.fori_loop(0, NC, logit_chunk, 0, unroll=False)


def _main_call(queries, query_positions, feats, wlist, BQ=128):
    Q = queries.shape[0]
    grid = Q // BQ
    consts = [jnp.asarray(_F2), jnp.asarray(_E), jnp.asarray(_GS),
              jnp.asarray(_GT), jnp.asarray(_S), jnp.asarray(_ST),
              jnp.asarray(_DTILE), jnp.asarray(_ONES)]
    in_specs = [
        pl.BlockSpec((BQ, D_MODEL), lambda i: (i, 0)),
        pl.BlockSpec((BQ, 2), lambda i: (i, 0)),
        pl.BlockSpec((BQ * KNBR, D_MODEL), lambda i: (i, 0)),
    ]
    for w in consts + wlist:
        nd = w.ndim
        in_specs.append(pl.BlockSpec(w.shape, lambda i, _nd=nd: (0,) * _nd))
    return pl.pallas_call(
        _main_body,
        grid=(grid,),
        out_shape=jax.ShapeDtypeStruct((Q * KNBR, 1), jnp.float32),
        in_specs=in_specs,
        out_specs=pl.BlockSpec((BQ * KNBR, 1), lambda i: (i, 0)),
        scratch_shapes=[
            pltpu.VMEM((BQ, D_MODEL), jnp.float32),
            pltpu.VMEM((BQ, D_MODEL), jnp.float32),
        ],
        compiler_params=pltpu.CompilerParams(
            dimension_semantics=("arbitrary",),
        ),
    )(queries, query_positions, feats, *consts, *wlist)


def kernel(queries, query_batch_offsets, query_positions, stacked_feature_map,
           level_spatial_shapes, params):
    Q = queries.shape[0]
    table = stacked_feature_map.reshape(-1, D_MODEL)
    idx2d = _prep_indices(query_positions, query_batch_offsets, level_spatial_shapes)
    feats = _sc_gather(table, idx2d.reshape(-1))
    wlist = []
    for i in range(N_LAYERS):
        wlist += [
            params[f"l{i}_ln1_g"].reshape(1, D_MODEL),
            params[f"l{i}_ln1_b"].reshape(1, D_MODEL),
            params[f"l{i}_Wq"][:, _PERM], params[f"l{i}_Wk"][:, _PERM],
            params[f"l{i}_Wv"], params[f"l{i}_Wo"],
            params[f"l{i}_ln2_g"].reshape(1, D_MODEL),
            params[f"l{i}_ln2_b"].reshape(1, D_MODEL),
            params[f"l{i}_W1"], params[f"l{i}_b1"].reshape(1, D_FF),
            params[f"l{i}_W2"], params[f"l{i}_b2"].reshape(1, D_MODEL),
        ]
    logits_col = _main_call(queries, query_positions, feats, wlist)
    return logits_col.reshape(Q, KNBR)
